# Initial kernel scaffold; baseline (speedup 1.0000x reference)
import jax
import jax.numpy as jnp
from jax.experimental import pallas as pl


def kernel(transcripts, fg_action_embedding, bg_embedding):
    raise NotImplementedError("write your pallas kernel here")


# speedup vs baseline: 3.3129x; 3.3129x over previous
import functools

import jax
import jax.numpy as jnp
from jax import lax
from jax.experimental import pallas as pl
from jax.experimental.pallas import tpu as pltpu
from jax.experimental.pallas import tpu_sc as plsc

P, C, D = 5, 100000, 64
B, T = 4096, 200
N = B * T

_ROWS = 2000
_NFG = C // _ROWS
_TABLE_ROWS = (_NFG + 1) * _ROWS


def _table_body(fg_ref, bg_ref, out_ref):
    j = pl.program_id(0)

    @pl.when(j < _NFG)
    def _fg():
        m = jnp.mean(fg_ref[...], axis=0)
        norm = jnp.sqrt(jnp.sum(m * m, axis=1, keepdims=True))
        out_ref[...] = m / jnp.maximum(norm, 1e-5)

    @pl.when(j == _NFG)
    def _bg():
        b = bg_ref[...]
        norm = jnp.sqrt(jnp.sum(b * b, axis=1, keepdims=True))
        out_ref[...] = jnp.broadcast_to(b / jnp.maximum(norm, 1e-5), (_ROWS, D))


def _build_table(fg, bg):
    return pl.pallas_call(
        _table_body,
        grid=(_NFG + 1,),
        in_specs=[
            pl.BlockSpec((P, _ROWS, D), lambda j: (0, jnp.minimum(j, _NFG - 1), 0)),
            pl.BlockSpec((1, D), lambda j: (0, 0)),
        ],
        out_specs=pl.BlockSpec((_ROWS, D), lambda j: (j, 0)),
        out_shape=jax.ShapeDtypeStruct((_TABLE_ROWS, D), jnp.float32),
    )(fg, bg)


_IDX_ROWS = N // 128


def _remap_body(t_ref, out_ref):
    t = t_ref[...]
    out_ref[...] = jnp.where(t == 0, C, t - 1)


def _remap_indices(transcripts):
    t = transcripts.astype(jnp.int32).reshape(_IDX_ROWS, 128)
    return pl.pallas_call(
        _remap_body,
        grid=(8,),
        in_specs=[pl.BlockSpec((_IDX_ROWS // 8, 128), lambda j: (j, 0))],
        out_specs=pl.BlockSpec((_IDX_ROWS // 8, 128), lambda j: (j, 0)),
        out_shape=jax.ShapeDtypeStruct((_IDX_ROWS, 128), jnp.int32),
    )(t)


_NC, _NS = 2, 16
_NW = _NC * _NS
_PER_W = N // _NW
_CHUNK = 128
_NCHUNK = _PER_W // _CHUNK
_K = 8
_NSUPER = _NCHUNK // _K


def _gather_body(table_hbm, idx_hbm, out_hbm, idx_v, rows_v, gsem):
    wid = lax.axis_index("s") * _NC + lax.axis_index("c")
    pltpu.sync_copy(idx_hbm.at[pl.ds(wid * _NCHUNK, _NCHUNK)], idx_v)
    base = wid * _PER_W

    def superstep(s, carry):
        cps = []
        for b in range(_K):
            cps.append(
                pltpu.async_copy(
                    table_hbm.at[idx_v.at[s * _K + b]],
                    rows_v.at[pl.ds(b * _CHUNK, _CHUNK)],
                    gsem,
                )
            )
        for cp in cps:
            cp.wait()
        pltpu.sync_copy(rows_v, out_hbm.at[pl.ds(base + s * (_K * _CHUNK), _K * _CHUNK)])
        return carry

    lax.fori_loop(0, _NSUPER, superstep, 0)


@functools.cache
def _make_gather():
    @functools.partial(
        pl.kernel,
        mesh=plsc.VectorSubcoreMesh(core_axis_name="c", subcore_axis_name="s"),
        out_type=jax.ShapeDtypeStruct((N, D), jnp.float32),
        compiler_params=pltpu.CompilerParams(use_tc_tiling_on_sc=False),
        scratch_types=[
            pltpu.VMEM((_NCHUNK, 128), jnp.int32),
            pltpu.VMEM((_K * _CHUNK, D), jnp.float32),
            pltpu.SemaphoreType.DMA,
        ],
    )
    def _gather_rows(table_hbm, idx_hbm, out_hbm, idx_v, rows_v, gsem):
        _gather_body(table_hbm, idx_hbm, out_hbm, idx_v, rows_v, gsem)

    return _gather_rows


def kernel(transcripts, fg_action_embedding, bg_embedding):
    table = _build_table(fg_action_embedding, bg_embedding)
    idx = _remap_indices(transcripts)
    out = _make_gather()(table, idx)
    return out.reshape(B, T, D)


# speedup vs baseline: 4.1349x; 1.2481x over previous
import functools

import jax
import jax.numpy as jnp
from jax import lax
from jax.experimental import pallas as pl
from jax.experimental.pallas import tpu as pltpu
from jax.experimental.pallas import tpu_sc as plsc

P, C, D = 5, 100000, 64
B, T = 4096, 200
N = B * T

_ROWS = 2048
_NFG = -(-C // _ROWS)
_BG_ROW = _NFG * _ROWS
_TABLE_ROWS = (_NFG + 1) * _ROWS


def _table_body(fg_ref, bg_ref, out_ref):
    j = pl.program_id(0)

    @pl.when(j < _NFG)
    def _fg():
        x = fg_ref[...]
        m = jnp.sum(x, axis=0) * (1.0 / P)
        norm = jnp.sqrt(jnp.sum(m * m, axis=0, keepdims=True))
        normed = m / jnp.maximum(norm, 1e-5)
        out_ref[...] = normed.T

    @pl.when(j == _NFG)
    def _bg():
        b = bg_ref[...]
        norm = jnp.sqrt(jnp.sum(b * b, axis=1, keepdims=True))
        out_ref[...] = jnp.broadcast_to(b / jnp.maximum(norm, 1e-5), (_ROWS, D))


def _build_table(fg, bg):
    fg_t = jnp.transpose(fg, (0, 2, 1))
    return pl.pallas_call(
        _table_body,
        grid=(_NFG + 1,),
        in_specs=[
            pl.BlockSpec((P, D, _ROWS), lambda j: (0, 0, jnp.minimum(j, _NFG - 1))),
            pl.BlockSpec((1, D), lambda j: (0, 0)),
        ],
        out_specs=pl.BlockSpec((_ROWS, D), lambda j: (j, 0)),
        out_shape=jax.ShapeDtypeStruct((_TABLE_ROWS, D), jnp.float32),
    )(fg_t, bg)


_IDX_ROWS = N // 128


def _remap_body(t_ref, out_ref):
    t = t_ref[...]
    out_ref[...] = jnp.where(t == 0, _BG_ROW, t - 1)


def _remap_indices(transcripts):
    t = transcripts.astype(jnp.int32).reshape(_IDX_ROWS, 128)
    return pl.pallas_call(
        _remap_body,
        grid=(8,),
        in_specs=[pl.BlockSpec((_IDX_ROWS // 8, 128), lambda j: (j, 0))],
        out_specs=pl.BlockSpec((_IDX_ROWS // 8, 128), lambda j: (j, 0)),
        out_shape=jax.ShapeDtypeStruct((_IDX_ROWS, 128), jnp.int32),
    )(t)


_NC, _NS = 2, 16
_NW = _NC * _NS
_BPW = B // _NW
_PER_W = _BPW * T
_SLABS = 4
_NSUPER = _BPW // _SLABS
_SPLIT = (104, 96)


def _gather_body(table_hbm, idx_hbm, out_hbm, idx_v, rows_v, gsem):
    wid = lax.axis_index("s") * _NC + lax.axis_index("c")
    pltpu.sync_copy(idx_hbm.at[pl.ds(wid * _PER_W, _PER_W)], idx_v)

    def superstep(s, carry):
        cps = []
        for i in range(_SLABS):
            off = 0
            for n in _SPLIT:
                cps.append(
                    pltpu.async_copy(
                        table_hbm.at[idx_v.at[pl.ds((s * _SLABS + i) * T + off, n)]],
                        rows_v.at[i, pl.ds(off, n)],
                        gsem,
                    )
                )
                off += n
        for cp in cps:
            cp.wait()
        pltpu.sync_copy(rows_v, out_hbm.at[pl.ds(wid * _BPW + s * _SLABS, _SLABS)])
        return carry

    lax.fori_loop(0, _NSUPER, superstep, 0)


@functools.cache
def _make_gather():
    @functools.partial(
        pl.kernel,
        mesh=plsc.VectorSubcoreMesh(core_axis_name="c", subcore_axis_name="s"),
        out_type=jax.ShapeDtypeStruct((B, T, D), jnp.float32),
        compiler_params=pltpu.CompilerParams(use_tc_tiling_on_sc=False),
        scratch_types=[
            pltpu.VMEM((_PER_W,), jnp.int32),
            pltpu.VMEM((_SLABS, T, D), jnp.float32),
            pltpu.SemaphoreType.DMA,
        ],
    )
    def _gather_rows(table_hbm, idx_hbm, out_hbm, idx_v, rows_v, gsem):
        _gather_body(table_hbm, idx_hbm, out_hbm, idx_v, rows_v, gsem)

    return _gather_rows


def kernel(transcripts, fg_action_embedding, bg_embedding):
    table = _build_table(fg_action_embedding, bg_embedding)
    idx = _remap_indices(transcripts).reshape(N)
    return _make_gather()(table, idx)


# speedup vs baseline: 4.3824x; 1.0599x over previous
import functools

import jax
import jax.numpy as jnp
from jax import lax
from jax.experimental import pallas as pl
from jax.experimental.pallas import tpu as pltpu
from jax.experimental.pallas import tpu_sc as plsc

P, C, D = 5, 100000, 64
B, T = 4096, 200
N = B * T

_ROWS = 2048
_NFG = -(-C // _ROWS)
_BG_ROW = _NFG * _ROWS
_TABLE_ROWS = (_NFG + 1) * _ROWS


def _table_body(fg_ref, bg_ref, out_ref):
    j = pl.program_id(0)

    @pl.when(j < _NFG)
    def _fg():
        x = fg_ref[...]
        m = jnp.sum(x, axis=0) * (1.0 / P)
        norm = jnp.sqrt(jnp.sum(m * m, axis=0, keepdims=True))
        normed = m / jnp.maximum(norm, 1e-5)
        out_ref[...] = normed.T

    @pl.when(j == _NFG)
    def _bg():
        b = bg_ref[...]
        norm = jnp.sqrt(jnp.sum(b * b, axis=1, keepdims=True))
        out_ref[...] = jnp.broadcast_to(b / jnp.maximum(norm, 1e-5), (_ROWS, D))


def _build_table(fg, bg):
    fg_t = jnp.transpose(fg, (0, 2, 1))
    return pl.pallas_call(
        _table_body,
        grid=(_NFG + 1,),
        in_specs=[
            pl.BlockSpec((P, D, _ROWS), lambda j: (0, 0, jnp.minimum(j, _NFG - 1))),
            pl.BlockSpec((1, D), lambda j: (0, 0)),
        ],
        out_specs=pl.BlockSpec((_ROWS, D), lambda j: (j, 0)),
        out_shape=jax.ShapeDtypeStruct((_TABLE_ROWS, D), jnp.float32),
    )(fg_t, bg)


def _remap_body(t_ref, out_ref):
    t = t_ref[...]
    out_ref[...] = jnp.where(t == 0, _BG_ROW, t - 1)


def _remap_indices(transcripts):
    t_t = jnp.transpose(transcripts.astype(jnp.int32))
    return pl.pallas_call(
        _remap_body,
        grid=(8,),
        in_specs=[pl.BlockSpec((T, B // 8), lambda j: (0, j))],
        out_specs=pl.BlockSpec((T, B // 8), lambda j: (0, j)),
        out_shape=jax.ShapeDtypeStruct((T, B), jnp.int32),
    )(t_t)


_NC, _NS = 2, 16
_NW = _NC * _NS
_BPW = B // _NW
_K = 8
_NSUPER = T // _K


def _gather_body(table_hbm, idx_hbm, out_hbm, idx_v, rows_v, gsem, ssem):
    wid = lax.axis_index("s") * _NC + lax.axis_index("c")
    b0 = wid * _BPW
    pltpu.sync_copy(idx_hbm.at[:, pl.ds(b0, _BPW)], idx_v)

    def superstep(s, carry):
        cps = []
        for i in range(_K):
            cps.append(
                pltpu.async_copy(
                    table_hbm.at[idx_v.at[s * _K + i]],
                    rows_v.at[i],
                    gsem,
                )
            )
        for cp in cps:
            cp.wait()
        sps = []
        for i in range(_K):
            sps.append(
                pltpu.async_copy(
                    rows_v.at[i],
                    out_hbm.at[s * _K + i, pl.ds(b0, _BPW)],
                    ssem,
                )
            )
        for sp in sps:
            sp.wait()
        return carry

    lax.fori_loop(0, _NSUPER, superstep, 0)


@functools.cache
def _make_gather():
    @functools.partial(
        pl.kernel,
        mesh=plsc.VectorSubcoreMesh(core_axis_name="c", subcore_axis_name="s"),
        out_type=jax.ShapeDtypeStruct((T, B, D), jnp.float32),
        compiler_params=pltpu.CompilerParams(use_tc_tiling_on_sc=False),
        scratch_types=[
            pltpu.VMEM((T, _BPW), jnp.int32),
            pltpu.VMEM((_K, _BPW, D), jnp.float32),
            pltpu.SemaphoreType.DMA,
            pltpu.SemaphoreType.DMA,
        ],
    )
    def _gather_rows(table_hbm, idx_hbm, out_hbm, idx_v, rows_v, gsem, ssem):
        _gather_body(table_hbm, idx_hbm, out_hbm, idx_v, rows_v, gsem, ssem)

    return _gather_rows


def kernel(transcripts, fg_action_embedding, bg_embedding):
    table = _build_table(fg_action_embedding, bg_embedding)
    idx = _remap_indices(transcripts)
    out = _make_gather()(table, idx)
    return jnp.transpose(out, (1, 0, 2))
